# baseline (speedup 1.0000x reference)
import jax
import jax.numpy as jnp
from jax import lax
from jax.experimental import pallas as pl
from jax.experimental.pallas import tpu as pltpu
from jax.experimental.pallas import tpu_sc as plsc

_FILTER = 9
_ALPHA = 20
_PEP_LEN = 15
_NUM_POCKET = 34
_OUT = _FILTER + _ALPHA - 1
_L = 16
_PAD = _FILTER - 1
_NPAIR = _NUM_POCKET // 2

_POC_OFF = _PEP_LEN * _ALPHA + _L + 4
_KER_OFF = _POC_OFF + 48
_BUF = _KER_OFF + _ALPHA * _L


def _body(buf_hbm, out_hbm, buf_v, xpad_v, row2_v, sem, semo0, semo1):
  wid = lax.axis_index("s")

  pltpu.async_copy(buf_hbm, buf_v, sem).wait()

  zeros = jnp.zeros((_L,), jnp.float32)
  lane = lax.iota(jnp.int32, _L)

  def do_pair(pair, slot, semo):
    for q in range(2):
      pocket = 2 * pair + q
      j1 = lax.rem(pocket, _PEP_LEN)
      j2 = lax.rem(pocket + 7, _PEP_LEN)
      o1 = j1 * _ALPHA
      o2 = j2 * _ALPHA
      a = buf_v[pl.ds(o1, _L)] + buf_v[pl.ds(o2, _L)]
      b = buf_v[pl.ds(o1 + _L, _L)] + buf_v[pl.ds(o2 + _L, _L)]
      b = jnp.where(lane < _ALPHA - _L, b, 0.0)
      xpad_v[pl.ds(0, _L)] = zeros
      xpad_v[pl.ds(_PAD, _L)] = a
      xpad_v[pl.ds(_PAD + _L, _L)] = b
      pid_f = plsc.load_gather(buf_v, [jnp.broadcast_to(_POC_OFF + pocket, (_L,))])
      pid = pid_f.astype(jnp.int32)
      acc0 = zeros
      acc1 = zeros
      for t in range(_FILTER):
        tap = plsc.load_gather(buf_v, [pid * _L + (_KER_OFF + t)])
        acc0 = acc0 + tap * xpad_v[pl.ds(_PAD - t, _L)]
        acc1 = acc1 + tap * xpad_v[pl.ds(_PAD + _L - t, _L)]
      base = slot * 4 * _L + q * _OUT
      row2_v[pl.ds(base, _L)] = acc0
      row2_v[pl.ds(base + _L, _L)] = acc1
    return pltpu.async_copy(
        row2_v.at[pl.ds(slot * 4 * _L, 2 * _OUT)],
        out_hbm.at[pl.ds(2 * _OUT * pair, 2 * _OUT)], semo)

  cp0 = do_pair(wid, 0, semo0)

  @pl.when(wid == 0)
  def _():
    do_pair(_NPAIR - 1, 1, semo1).wait()

  cp0.wait()


@jax.jit
def kernel(peptide_encoding, pocket_encoding, kernels):
  poc_f = jnp.pad(pocket_encoding, (0, 48 - _NUM_POCKET)).astype(jnp.float32)
  buf = jnp.concatenate([
      peptide_encoding.reshape(-1),
      jnp.zeros((_L + 4,), jnp.float32),
      poc_f,
      jnp.pad(kernels, ((0, 0), (0, _L - _FILTER))).reshape(-1),
  ])

  out = pl.kernel(
      _body,
      out_type=jax.ShapeDtypeStruct((_NUM_POCKET * _OUT,), jnp.float32),
      mesh=plsc.VectorSubcoreMesh(
          core_axis_name="c", subcore_axis_name="s", num_cores=1),
      compiler_params=pltpu.CompilerParams(needs_layout_passes=False),
      scratch_types=[
          pltpu.VMEM((_BUF,), jnp.float32),
          pltpu.VMEM((_ALPHA + 2 * _PAD + 4,), jnp.float32),
          pltpu.VMEM((8 * _L,), jnp.float32),
          pltpu.SemaphoreType.DMA,
          pltpu.SemaphoreType.DMA,
          pltpu.SemaphoreType.DMA,
      ],
  )(buf)
  return out.reshape(_NUM_POCKET, _OUT)


# speedup vs baseline: 1.0075x; 1.0075x over previous
import jax
import jax.numpy as jnp
from jax import lax
from jax.experimental import pallas as pl
from jax.experimental.pallas import tpu as pltpu
from jax.experimental.pallas import tpu_sc as plsc

_FILTER = 9
_ALPHA = 20
_PEP_LEN = 15
_NUM_POCKET = 34
_OUT = _FILTER + _ALPHA - 1
_L = 16
_PAD = _FILTER - 1
_NPAIR = _NUM_POCKET // 2

_POC_OFF = _PEP_LEN * _ALPHA + _L + 4
_KER_OFF = _POC_OFF + 48
_BUF = _KER_OFF + _ALPHA * _FILTER + 12


def _body(buf_hbm, out_hbm, buf_v, xpad_v, row2_v, sem):
  wid = lax.axis_index("s")

  pltpu.async_copy(buf_hbm, buf_v, sem).wait()

  zeros = jnp.zeros((_L,), jnp.float32)
  lane = lax.iota(jnp.int32, _L)

  def do_pair(k, carry):
    pair = wid + 16 * k
    for q in range(2):
      pocket = 2 * pair + q
      j1 = lax.rem(pocket, _PEP_LEN)
      j2 = lax.rem(pocket + 7, _PEP_LEN)
      o1 = j1 * _ALPHA
      o2 = j2 * _ALPHA
      a = buf_v[pl.ds(o1, _L)] + buf_v[pl.ds(o2, _L)]
      b = buf_v[pl.ds(o1 + _L, _L)] + buf_v[pl.ds(o2 + _L, _L)]
      b = jnp.where(lane < _ALPHA - _L, b, 0.0)
      xpad_v[pl.ds(0, _L)] = zeros
      xpad_v[pl.ds(_PAD, _L)] = a
      xpad_v[pl.ds(_PAD + _L, _L)] = b
      pid_f = plsc.load_gather(buf_v, [jnp.broadcast_to(_POC_OFF + pocket, (_L,))])
      pid = pid_f.astype(jnp.int32)
      acc0 = zeros
      acc1 = zeros
      for t in range(_FILTER):
        tap = plsc.load_gather(buf_v, [pid * _FILTER + (_KER_OFF + t)])
        acc0 = acc0 + tap * xpad_v[pl.ds(_PAD - t, _L)]
        acc1 = acc1 + tap * xpad_v[pl.ds(_PAD + _L - t, _L)]
      base = q * _OUT
      row2_v[pl.ds(base, _L)] = acc0
      row2_v[pl.ds(base + _L, _L)] = acc1
    pltpu.sync_copy(row2_v.at[pl.ds(0, 2 * _OUT)],
                    out_hbm.at[pl.ds(2 * _OUT * pair, 2 * _OUT)])
    return carry

  n_iters = 1 + (wid == 0).astype(jnp.int32)
  lax.fori_loop(0, n_iters, do_pair, 0)


@jax.jit
def kernel(peptide_encoding, pocket_encoding, kernels):
  poc_f = jnp.pad(pocket_encoding, (0, 48 - _NUM_POCKET)).astype(jnp.float32)
  buf = jnp.concatenate([
      peptide_encoding.reshape(-1),
      jnp.zeros((_L + 4,), jnp.float32),
      poc_f,
      kernels.reshape(-1),
      jnp.zeros((12,), jnp.float32),
  ])

  out = pl.kernel(
      _body,
      out_type=jax.ShapeDtypeStruct((_NUM_POCKET * _OUT,), jnp.float32),
      mesh=plsc.VectorSubcoreMesh(
          core_axis_name="c", subcore_axis_name="s", num_cores=1),
      compiler_params=pltpu.CompilerParams(needs_layout_passes=False),
      scratch_types=[
          pltpu.VMEM((_BUF,), jnp.float32),
          pltpu.VMEM((_ALPHA + 2 * _PAD + 4,), jnp.float32),
          pltpu.VMEM((4 * _L,), jnp.float32),
          pltpu.SemaphoreType.DMA,
      ],
  )(buf)
  return out.reshape(_NUM_POCKET, _OUT)
